# baseline (speedup 1.0000x reference)
import jax
import jax.numpy as jnp
from jax.experimental import pallas as pl
from jax.experimental.pallas import tpu as pltpu

ROW_BLOCK = 200


def _kgagg_body(a_ref, e_ref, wsT_ref, wnT_ref, b_ref, out_ref):
    i = pl.program_id(0)
    neigh = jnp.dot(a_ref[...], e_ref[...], preferred_element_type=jnp.float32)
    neigh = jnp.dot(neigh, wnT_ref[...], preferred_element_type=jnp.float32)
    e_blk = e_ref[pl.ds(i * ROW_BLOCK, ROW_BLOCK), :]
    self_t = jnp.dot(e_blk, wsT_ref[...], preferred_element_type=jnp.float32)
    x = self_t + neigh + b_ref[...]
    out_ref[...] = jnp.where(x >= 0.0, x, 0.01 * x)


def kernel(entity_embs, adj_matrix, W_self, b_self, W_neigh, b_neigh):
    n, d_in = entity_embs.shape
    d_out = W_self.shape[0]
    bias = (b_self + b_neigh).reshape(1, d_out)
    return pl.pallas_call(
        _kgagg_body,
        grid=(n // ROW_BLOCK,),
        in_specs=[
            pl.BlockSpec((ROW_BLOCK, n), lambda i: (i, 0)),
            pl.BlockSpec((n, d_in), lambda i: (0, 0)),
            pl.BlockSpec((d_in, d_out), lambda i: (0, 0)),
            pl.BlockSpec((d_in, d_out), lambda i: (0, 0)),
            pl.BlockSpec((1, d_out), lambda i: (0, 0)),
        ],
        out_specs=pl.BlockSpec((ROW_BLOCK, d_out), lambda i: (i, 0)),
        out_shape=jax.ShapeDtypeStruct((n, d_out), jnp.float32),
        compiler_params=pltpu.CompilerParams(
            dimension_semantics=("arbitrary",),
        ),
    )(adj_matrix, entity_embs, W_self.T, W_neigh.T, bias)


# speedup vs baseline: 1.0379x; 1.0379x over previous
import jax
import jax.numpy as jnp
from jax.experimental import pallas as pl
from jax.experimental.pallas import tpu as pltpu

ROW_BLOCK = 400


def _kgagg_body(a_ref, e_ref, wsT_ref, wnT_ref, b_ref, out_ref):
    i = pl.program_id(0)
    neigh = jnp.dot(a_ref[...], e_ref[...], preferred_element_type=jnp.float32)
    neigh = jnp.dot(neigh, wnT_ref[...], preferred_element_type=jnp.float32)
    e_blk = e_ref[pl.ds(i * ROW_BLOCK, ROW_BLOCK), :]
    self_t = jnp.dot(e_blk, wsT_ref[...], preferred_element_type=jnp.float32)
    x = self_t + neigh + b_ref[...]
    out_ref[...] = jnp.where(x >= 0.0, x, 0.01 * x)


def kernel(entity_embs, adj_matrix, W_self, b_self, W_neigh, b_neigh):
    n, d_in = entity_embs.shape
    d_out = W_self.shape[0]
    bias = (b_self + b_neigh).reshape(1, d_out)
    return pl.pallas_call(
        _kgagg_body,
        grid=(n // ROW_BLOCK,),
        in_specs=[
            pl.BlockSpec((ROW_BLOCK, n), lambda i: (i, 0)),
            pl.BlockSpec((n, d_in), lambda i: (0, 0)),
            pl.BlockSpec((d_in, d_out), lambda i: (0, 0)),
            pl.BlockSpec((d_in, d_out), lambda i: (0, 0)),
            pl.BlockSpec((1, d_out), lambda i: (0, 0)),
        ],
        out_specs=pl.BlockSpec((ROW_BLOCK, d_out), lambda i: (i, 0)),
        out_shape=jax.ShapeDtypeStruct((n, d_out), jnp.float32),
        compiler_params=pltpu.CompilerParams(
            dimension_semantics=("arbitrary",),
        ),
    )(adj_matrix, entity_embs, W_self.T, W_neigh.T, bias)
